# baseline (speedup 1.0000x reference)
import functools

import jax
import jax.numpy as jnp
from jax import lax
from jax.experimental import pallas as pl
from jax.experimental.pallas import tpu as pltpu
from jax.experimental.pallas import tpu_sc as plsc

B = 16384
D = 64
NW = 32
BPW = B // NW
CH = 128
NCH = BPW // CH
RING = 8


@functools.cache
def _sc_user_fn():
    mesh = plsc.VectorSubcoreMesh(core_axis_name="c", subcore_axis_name="s")

    @functools.partial(
        pl.kernel,
        mesh=mesh,
        out_type=jax.ShapeDtypeStruct((B, D), jnp.float32),
        scratch_types=(
            pltpu.VMEM((BPW,), jnp.int32),
            pltpu.VMEM((RING, D, 128), jnp.float32),
            pltpu.VMEM((CH, D), jnp.float32),
            pltpu.SemaphoreType.DMA,
        ),
        compiler_params=pltpu.CompilerParams(use_tc_tiling_on_sc=True,
                                             needs_layout_passes=False),
    )
    def _sc_user(uids_hbm, utabt_hbm, uout_hbm, uidx_vm, ring, outc, sem):
        wid = lax.axis_index("s") * 2 + lax.axis_index("c")
        base = wid * BPW
        pltpu.sync_copy(uids_hbm.at[pl.ds(base, BPW)], uidx_vm)

        rows16 = [lax.iota(jnp.int32, 16) + (16 * k) for k in range(D // 16)]

        for c in range(NCH):
            def ugroup(g, _, c=c):
                goff = c * CH + g * 16
                vec = uidx_vm[pl.ds(goff, 16)]
                for h in range(2):
                    copies = []
                    idxs = []
                    for l in range(RING):
                        idx = vec[h * RING + l]
                        col0 = pl.multiple_of((idx >> 7) * 128, 128)
                        copies.append(pltpu.make_async_copy(
                            utabt_hbm.at[:, pl.ds(col0, 128)], ring.at[l], sem))
                        idxs.append(idx)
                    for cp in copies:
                        cp.start()
                    for cp in copies:
                        cp.wait()
                    for l in range(RING):
                        lc = jnp.full((16,), idxs[l] & 127, jnp.int32)
                        row = g * 16 + h * RING + l
                        for k in range(D // 16):
                            seg = plsc.load_gather(ring.at[l], [rows16[k], lc])
                            outc[row, pl.ds(16 * k, 16)] = seg
                return 0

            lax.fori_loop(0, CH // 16, ugroup, 0)
            pltpu.sync_copy(outc, uout_hbm.at[pl.ds(base + c * CH, CH)])

    return _sc_user


@functools.cache
def _sc_item_fn():
    mesh = plsc.VectorSubcoreMesh(core_axis_name="c", subcore_axis_name="s")

    @functools.partial(
        pl.kernel,
        mesh=mesh,
        out_type=jax.ShapeDtypeStruct((B, D), jnp.float32),
        scratch_types=(
            pltpu.VMEM((BPW,), jnp.int32),
            pltpu.VMEM((CH, D), jnp.float32),
            pltpu.SemaphoreType.DMA,
        ),
        compiler_params=pltpu.CompilerParams(use_tc_tiling_on_sc=True,
                                             needs_layout_passes=False),
    )
    def _sc_item(iids_hbm, itab_hbm, iout_hbm, iidx_vm, outc, sem):
        wid = lax.axis_index("s") * 2 + lax.axis_index("c")
        base = wid * BPW
        pltpu.sync_copy(iids_hbm.at[pl.ds(base, BPW)], iidx_vm)

        for c in range(NCH):
            def igroup(g, _, c=c):
                goff = c * CH + g * 16
                vec = iidx_vm[pl.ds(goff, 16)]
                for l in range(16):
                    pltpu.make_async_copy(itab_hbm.at[vec[l]],
                                          outc.at[g * 16 + l], sem).start()
                return 0

            lax.fori_loop(0, CH // 16, igroup, 0)
            pltpu.make_async_copy(itab_hbm.at[pl.ds(0, CH)], outc, sem).wait()
            pltpu.sync_copy(outc, iout_hbm.at[pl.ds(base + c * CH, CH)])

    return _sc_item


BM = 2048


def _mlp_body(u_ref, i_ref, w1u_ref, w1i_ref, b1_ref, w2_ref, b2_ref,
              w3_ref, b3_ref, out_ref):
    h1 = jnp.dot(u_ref[...], w1u_ref[...], preferred_element_type=jnp.float32)
    h1 += jnp.dot(i_ref[...], w1i_ref[...], preferred_element_type=jnp.float32)
    h1 = jnp.maximum(h1 + b1_ref[...], 0.0)
    h2 = jnp.dot(h1, w2_ref[...], preferred_element_type=jnp.float32)
    h2 = jnp.maximum(h2 + b2_ref[...], 0.0)
    out_ref[...] = jnp.sum(h2 * w3_ref[...], axis=1, keepdims=True) + b3_ref[...]


def _mlp(u_emb, i_emb, w1u, w1i, b1, w2, b2, w3, b3):
    grid = (B // BM,)
    full = lambda r, c: pl.BlockSpec((r, c), lambda m: (0, 0))
    return pl.pallas_call(
        _mlp_body,
        grid=grid,
        in_specs=[
            pl.BlockSpec((BM, D), lambda m: (m, 0)),
            pl.BlockSpec((BM, D), lambda m: (m, 0)),
            full(D, D),
            full(D, D),
            full(1, D),
            full(D, 32),
            full(1, 32),
            full(1, 32),
            full(1, 1),
        ],
        out_specs=pl.BlockSpec((BM, 1), lambda m: (m, 0)),
        out_shape=jax.ShapeDtypeStruct((B, 1), jnp.float32),
    )(u_emb, i_emb, w1u, w1i, b1, w2, b2, w3, b3)


def kernel(user_ids, item_ids, user_table, item_table, W1, b1, W2, b2, W3, b3):
    uids = user_ids.astype(jnp.int32)
    iids = item_ids.astype(jnp.int32)
    u_emb = _sc_user_fn()(uids, user_table.T)
    i_emb = _sc_item_fn()(iids, item_table)
    w1u = W1[:, :D].T
    w1i = W1[:, D:].T
    return _mlp(u_emb, i_emb, w1u, w1i, b1.reshape(1, D),
                W2.T, b2.reshape(1, 32), W3.reshape(1, 32), b3.reshape(1, 1))


# speedup vs baseline: 1.0173x; 1.0173x over previous
import functools

import jax
import jax.numpy as jnp
from jax import lax
from jax.experimental import pallas as pl
from jax.experimental.pallas import tpu as pltpu
from jax.experimental.pallas import tpu_sc as plsc

B = 16384
D = 64
NW = 32
BPW = B // NW
CH = 128
NCH = BPW // CH
RING = 8


@functools.cache
def _sc_gather_fn():
    mesh = plsc.VectorSubcoreMesh(core_axis_name="c", subcore_axis_name="s")

    @functools.partial(
        pl.kernel,
        mesh=mesh,
        out_type=(
            jax.ShapeDtypeStruct((B, D), jnp.float32),
            jax.ShapeDtypeStruct((B, D), jnp.float32),
        ),
        scratch_types=(
            pltpu.VMEM((BPW,), jnp.int32),
            pltpu.VMEM((BPW,), jnp.int32),
            pltpu.VMEM((RING, D, 128), jnp.float32),
            pltpu.VMEM((CH, D), jnp.float32),
            pltpu.SemaphoreType.DMA,
        ),
        compiler_params=pltpu.CompilerParams(use_tc_tiling_on_sc=True,
                                             needs_layout_passes=False),
    )
    def _sc_gather(uids_hbm, iids_hbm, utabt_hbm, itab_hbm, uout_hbm, iout_hbm,
                   uidx_vm, iidx_vm, ring, outc, sem):
        wid = lax.axis_index("s") * 2 + lax.axis_index("c")
        base = wid * BPW
        pltpu.sync_copy(uids_hbm.at[pl.ds(base, BPW)], uidx_vm)
        pltpu.sync_copy(iids_hbm.at[pl.ds(base, BPW)], iidx_vm)

        rows16 = [lax.iota(jnp.int32, 16) + (16 * k) for k in range(D // 16)]
        NT = D // 8

        def uchunk(c, _):
            def ugroup(g, _):
                goff = c * CH + g * 16
                vec = uidx_vm[pl.ds(goff, 16)]
                for h in range(2):
                    copies = []
                    idxs = []
                    for l in range(RING):
                        idx = vec[h * RING + l]
                        col0 = pl.multiple_of((idx >> 7) * 128, 128)
                        for k in range(NT):
                            copies.append(pltpu.make_async_copy(
                                utabt_hbm.at[pl.ds(8 * k, 8), pl.ds(col0, 128)],
                                ring.at[l, pl.ds(8 * k, 8)], sem))
                        idxs.append(idx)
                    for cp in copies:
                        cp.start()
                    for cp in copies:
                        cp.wait()
                    for l in range(RING):
                        lc = jnp.full((16,), idxs[l] & 127, jnp.int32)
                        row = g * 16 + h * RING + l
                        for k in range(D // 16):
                            seg = plsc.load_gather(ring.at[l], [rows16[k], lc])
                            outc[row, pl.ds(16 * k, 16)] = seg
                return 0

            lax.fori_loop(0, CH // 16, ugroup, 0)
            pltpu.sync_copy(outc, uout_hbm.at[pl.ds(base + c * CH, CH)])
            return 0

        lax.fori_loop(0, NCH, uchunk, 0)

        def ichunk(c, _):
            def igroup(g, _):
                goff = c * CH + g * 16
                vec = iidx_vm[pl.ds(goff, 16)]
                for l in range(16):
                    pltpu.make_async_copy(itab_hbm.at[vec[l]],
                                          outc.at[g * 16 + l], sem).start()
                return 0

            lax.fori_loop(0, CH // 16, igroup, 0)
            pltpu.make_async_copy(itab_hbm.at[pl.ds(0, CH)], outc, sem).wait()
            pltpu.sync_copy(outc, iout_hbm.at[pl.ds(base + c * CH, CH)])
            return 0

        lax.fori_loop(0, NCH, ichunk, 0)

    return _sc_gather


BM = 2048


def _mlp_body(u_ref, i_ref, w1u_ref, w1i_ref, b1_ref, w2_ref, b2_ref,
              w3_ref, b3_ref, out_ref):
    h1 = jnp.dot(u_ref[...], w1u_ref[...], preferred_element_type=jnp.float32)
    h1 += jnp.dot(i_ref[...], w1i_ref[...], preferred_element_type=jnp.float32)
    h1 = jnp.maximum(h1 + b1_ref[...], 0.0)
    h2 = jnp.dot(h1, w2_ref[...], preferred_element_type=jnp.float32)
    h2 = jnp.maximum(h2 + b2_ref[...], 0.0)
    out_ref[...] = jnp.sum(h2 * w3_ref[...], axis=1, keepdims=True) + b3_ref[...]


def _mlp(u_emb, i_emb, w1u, w1i, b1, w2, b2, w3, b3):
    grid = (B // BM,)
    full = lambda r, c: pl.BlockSpec((r, c), lambda m: (0, 0))
    return pl.pallas_call(
        _mlp_body,
        grid=grid,
        in_specs=[
            pl.BlockSpec((BM, D), lambda m: (m, 0)),
            pl.BlockSpec((BM, D), lambda m: (m, 0)),
            full(D, D),
            full(D, D),
            full(1, D),
            full(D, 32),
            full(1, 32),
            full(1, 32),
            full(1, 1),
        ],
        out_specs=pl.BlockSpec((BM, 1), lambda m: (m, 0)),
        out_shape=jax.ShapeDtypeStruct((B, 1), jnp.float32),
    )(u_emb, i_emb, w1u, w1i, b1, w2, b2, w3, b3)


def kernel(user_ids, item_ids, user_table, item_table, W1, b1, W2, b2, W3, b3):
    uids = user_ids.astype(jnp.int32)
    iids = item_ids.astype(jnp.int32)
    u_emb, i_emb = _sc_gather_fn()(uids, iids, user_table.T, item_table)
    w1u = W1[:, :D].T
    w1i = W1[:, D:].T
    return _mlp(u_emb, i_emb, w1u, w1i, b1.reshape(1, D),
                W2.T, b2.reshape(1, 32), W3.reshape(1, 32), b3.reshape(1, 1))
